# Initial kernel scaffold; baseline (speedup 1.0000x reference)
import jax
import jax.numpy as jnp
from jax.experimental import pallas as pl


def kernel(p, x, o, edges, boundary, Wq, bq, Wk, bk, Wv, bv, Wp1, bp1, g_p, b_p, Wp2, bp2, g_w1, b_w1, Ww1, bw1, g_w2, b_w2, Ww2, bw2):
    raise NotImplementedError("write your pallas kernel here")


# speedup vs baseline: 4.9370x; 4.9370x over previous
import functools

import jax
import jax.numpy as jnp
from jax import lax
from jax.experimental import pallas as pl
from jax.experimental.pallas import tpu as pltpu
from jax.experimental.pallas import tpu_sc as plsc

N = 10000
K = 16
C = 128
CS = 16
SH = 8
RQ = 200
RB = 400
M = N * K

_PC = functools.partial(pl.pallas_call)


def _knn_body(pq_ref, pt_ref, x_ref, wk_ref, bk_ref, wv_ref, bv_ref,
              idx_ref, xk_ref, xv_ref, d_ref):
    pq = pq_ref[...]
    pt = pt_ref[...]
    sqq = jnp.sum(pq * pq, axis=1, keepdims=True)
    sqa = jnp.sum(pt * pt, axis=0, keepdims=True)
    cross = _dot_bf(pq, pt)
    d_ref[...] = sqq + sqa - 2.0 * cross

    lanes = lax.broadcasted_iota(jnp.int32, (RQ, N), 1)
    for t in range(K):
        d = d_ref[...]
        m = jnp.min(d, axis=1, keepdims=True)
        sel = jnp.min(jnp.where(d == m, lanes, N), axis=1, keepdims=True)
        idx_ref[:, t:t + 1] = sel
        d_ref[...] = jnp.where(lanes == sel, jnp.float32(jnp.inf), d)

    x = x_ref[...]
    xk_ref[...] = _dot_bf(x, wk_ref[...]) + bk_ref[...]
    xv_ref[...] = _dot_bf(x, wv_ref[...]) + bv_ref[...]


def _knn(p, pT, x, Wk, bk2, Wv, bv2):
    return _PC(
        _knn_body,
        grid=(N // RQ,),
        in_specs=[
            pl.BlockSpec((RQ, 3), lambda i: (i, 0)),
            pl.BlockSpec((3, N), lambda i: (0, 0)),
            pl.BlockSpec((RQ, C), lambda i: (i, 0)),
            pl.BlockSpec((C, C), lambda i: (0, 0)),
            pl.BlockSpec((1, C), lambda i: (0, 0)),
            pl.BlockSpec((C, C), lambda i: (0, 0)),
            pl.BlockSpec((1, C), lambda i: (0, 0)),
        ],
        out_specs=[
            pl.BlockSpec((RQ, K), lambda i: (i, 0)),
            pl.BlockSpec((RQ, C), lambda i: (i, 0)),
            pl.BlockSpec((RQ, C), lambda i: (i, 0)),
        ],
        out_shape=[
            jax.ShapeDtypeStruct((N, K), jnp.int32),
            jax.ShapeDtypeStruct((N, C), jnp.float32),
            jax.ShapeDtypeStruct((N, C), jnp.float32),
        ],
        scratch_shapes=[pltpu.VMEM((RQ, N), jnp.float32)],
    )(p, pT, x, Wk, bk2, Wv, bv2)


_NW = 32
_BPW = M // _NW
_CH = 200
_NCH = _BPW // _CH


def _sc_gather_body(xk_hbm, xv_hbm, pp_hbm, idx_hbm,
                    gk_hbm, gv_hbm, gp_hbm,
                    idx_v, rows_v, prow_v, sem):
    wid = lax.axis_index("s") * 2 + lax.axis_index("c")
    base = wid * _BPW
    for i in range(_NCH):
        off = base + i * _CH
        pltpu.sync_copy(idx_hbm.at[pl.ds(off, _CH)], idx_v)
        pltpu.async_copy(xk_hbm.at[idx_v], rows_v, sem).wait()
        pltpu.sync_copy(rows_v, gk_hbm.at[pl.ds(off, _CH)])
        pltpu.async_copy(xv_hbm.at[idx_v], rows_v, sem).wait()
        pltpu.sync_copy(rows_v, gv_hbm.at[pl.ds(off, _CH)])
        pltpu.async_copy(pp_hbm.at[idx_v], prow_v, sem).wait()
        pltpu.sync_copy(prow_v, gp_hbm.at[pl.ds(off, _CH)])


def _sc_gather_call(x_k, x_v, p_pad, idx_flat):
    mesh = plsc.VectorSubcoreMesh(core_axis_name="c", subcore_axis_name="s")
    fn = pl.kernel(
        _sc_gather_body,
        mesh=mesh,
        out_type=[
            jax.ShapeDtypeStruct((M, C), jnp.float32),
            jax.ShapeDtypeStruct((M, C), jnp.float32),
            jax.ShapeDtypeStruct((M, C), jnp.float32),
        ],
        scratch_types=[
            pltpu.VMEM((_CH,), jnp.int32),
            pltpu.VMEM((_CH, C), jnp.float32),
            pltpu.VMEM((_CH, C), jnp.float32),
            pltpu.SemaphoreType.DMA,
        ],
    )
    return fn(x_k, x_v, p_pad, idx_flat)


def _dot_bf(a, b):
    return jnp.dot(a.astype(jnp.bfloat16), b.astype(jnp.bfloat16),
                   preferred_element_type=jnp.float32)


def _mm3(a2, w_ref, b_ref):
    return _dot_bf(a2, w_ref[...]) + b_ref[...]


def _p_r(gx3, wp1_ref, bp1_ref, s1_ref, t1_ref, wp2_ref, bp2_ref):
    gxyz = gx3[:, :, 0:3].reshape(RB * K, 3)
    pr0 = _mm3(gxyz, wp1_ref, bp1_ref)
    prb = jnp.maximum(pr0 * s1_ref[...] + t1_ref[...], 0.0)
    return _mm3(prb, wp2_ref, bp2_ref)


def _bn1_body(gp_ref, p_ref, wp1_ref, bp1_ref, gx_ref, st_ref):
    gxyz3 = gp_ref[...][:, :, 0:3] - p_ref[...][:, None, :]
    gx_ref[...] = jnp.concatenate(
        [gxyz3, jnp.zeros((RB, K, 13), jnp.float32)], axis=2)
    pr0 = _mm3(gxyz3.reshape(RB * K, 3), wp1_ref, bp1_ref)
    s = jnp.sum(pr0, axis=0, keepdims=True)
    q = jnp.sum(pr0 * pr0, axis=0, keepdims=True)

    @pl.when(pl.program_id(0) == 0)
    def _():
        st_ref[...] = jnp.zeros_like(st_ref)

    st_ref[0:1, 0:3] += s
    st_ref[1:2, 0:3] += q


def _bn1_stats(gp, p, Wp1, bp12):
    return _PC(
        _bn1_body,
        grid=(N // RB,),
        in_specs=[
            pl.BlockSpec((RB, K, C), lambda i: (i, 0, 0)),
            pl.BlockSpec((RB, 3), lambda i: (i, 0)),
            pl.BlockSpec((3, 3), lambda i: (0, 0)),
            pl.BlockSpec((1, 3), lambda i: (0, 0)),
        ],
        out_specs=[
            pl.BlockSpec((RB, K, 16), lambda i: (i, 0, 0)),
            pl.BlockSpec((2, 128), lambda i: (0, 0)),
        ],
        out_shape=[
            jax.ShapeDtypeStruct((N, K, 16), jnp.float32),
            jax.ShapeDtypeStruct((2, 128), jnp.float32),
        ],
    )(gp, p, Wp1, bp12)


def _w1_body(gx_ref, gk_ref, x_ref, wq_ref, bq_ref,
             wp1_ref, bp1_ref, s1_ref, t1_ref, wp2_ref, bp2_ref,
             w1_ref, st_ref):
    xq = _dot_bf(x_ref[...], wq_ref[...]) + bq_ref[...]
    p_r = _p_r(gx_ref[...], wp1_ref, bp1_ref,
               s1_ref, t1_ref, wp2_ref, bp2_ref)
    gk2 = gk_ref[...].reshape(RB * K, C)
    xqr = jnp.broadcast_to(xq[:, None, :], (RB, K, C)).reshape(RB * K, C)
    w1 = gk2 - xqr + p_r
    w1_ref[...] = w1.reshape(RB, K, C)

    @pl.when(pl.program_id(0) == 0)
    def _():
        st_ref[...] = jnp.zeros_like(st_ref)

    st_ref[0:1, :] += jnp.sum(w1, axis=0, keepdims=True)
    st_ref[1:2, :] += jnp.sum(w1 * w1, axis=0, keepdims=True)


def _w1_pass(gx, gk, x, Wq, bq2, Wp1, bp12, s1, t1, Wp2, bp22):
    return _PC(
        _w1_body,
        grid=(N // RB,),
        in_specs=[
            pl.BlockSpec((RB, K, 16), lambda i: (i, 0, 0)),
            pl.BlockSpec((RB, K, C), lambda i: (i, 0, 0)),
            pl.BlockSpec((RB, C), lambda i: (i, 0)),
            pl.BlockSpec((C, C), lambda i: (0, 0)),
            pl.BlockSpec((1, C), lambda i: (0, 0)),
            pl.BlockSpec((3, 3), lambda i: (0, 0)),
            pl.BlockSpec((1, 3), lambda i: (0, 0)),
            pl.BlockSpec((1, 3), lambda i: (0, 0)),
            pl.BlockSpec((1, 3), lambda i: (0, 0)),
            pl.BlockSpec((3, C), lambda i: (0, 0)),
            pl.BlockSpec((1, C), lambda i: (0, 0)),
        ],
        out_specs=[
            pl.BlockSpec((RB, K, C), lambda i: (i, 0, 0)),
            pl.BlockSpec((2, C), lambda i: (0, 0)),
        ],
        out_shape=[
            jax.ShapeDtypeStruct((N, K, C), jnp.float32),
            jax.ShapeDtypeStruct((2, C), jnp.float32),
        ],
    )(gx, gk, x, Wq, bq2, Wp1, bp12, s1, t1, Wp2, bp22)


def _w2_body(w1_ref, s2_ref, t2_ref, ww1_ref, bw1_ref, w2_ref, st_ref):
    h = jnp.maximum(w1_ref[...].reshape(RB * K, C) * s2_ref[...]
                    + t2_ref[...], 0.0)
    w2 = _dot_bf(h, ww1_ref[...]) + bw1_ref[...]
    w2_ref[...] = w2.reshape(RB, K, CS)

    @pl.when(pl.program_id(0) == 0)
    def _():
        st_ref[...] = jnp.zeros_like(st_ref)

    st_ref[0:1, 0:CS] += jnp.sum(w2, axis=0, keepdims=True)
    st_ref[1:2, 0:CS] += jnp.sum(w2 * w2, axis=0, keepdims=True)


def _w2_pass(w1, s2, t2, Ww1, bw12):
    return _PC(
        _w2_body,
        grid=(N // RB,),
        in_specs=[
            pl.BlockSpec((RB, K, C), lambda i: (i, 0, 0)),
            pl.BlockSpec((1, C), lambda i: (0, 0)),
            pl.BlockSpec((1, C), lambda i: (0, 0)),
            pl.BlockSpec((C, CS), lambda i: (0, 0)),
            pl.BlockSpec((1, CS), lambda i: (0, 0)),
        ],
        out_specs=[
            pl.BlockSpec((RB, K, CS), lambda i: (i, 0, 0)),
            pl.BlockSpec((2, 128), lambda i: (0, 0)),
        ],
        out_shape=[
            jax.ShapeDtypeStruct((N, K, CS), jnp.float32),
            jax.ShapeDtypeStruct((2, 128), jnp.float32),
        ],
    )(w1, s2, t2, Ww1, bw12)


def _out_body(w2_ref, gx_ref, gv_ref, s3_ref, t3_ref, ww2_ref, bw2_ref,
              wp1_ref, bp1_ref, s1_ref, t1_ref, wp2_ref, bp2_ref, out_ref):
    h2 = jnp.maximum(w2_ref[...].reshape(RB * K, CS) * s3_ref[...]
                     + t3_ref[...], 0.0)
    a = (_dot_bf(h2, ww2_ref[...]) + bw2_ref[...]).reshape(RB, K, CS)
    mx = jnp.max(a, axis=1, keepdims=True)
    e = jnp.exp(a - mx)
    w = e / jnp.sum(e, axis=1, keepdims=True)
    wfull = jnp.concatenate([w] * SH, axis=2)
    p_r = _p_r(gx_ref[...], wp1_ref, bp1_ref,
               s1_ref, t1_ref, wp2_ref, bp2_ref).reshape(RB, K, C)
    out_ref[...] = jnp.sum((gv_ref[...] + p_r) * wfull, axis=1)


def _out_pass(w2, gx, gv, s3, t3, Ww2, bw22, Wp1, bp12, s1, t1, Wp2, bp22):
    return _PC(
        _out_body,
        grid=(N // RB,),
        in_specs=[
            pl.BlockSpec((RB, K, CS), lambda i: (i, 0, 0)),
            pl.BlockSpec((RB, K, 16), lambda i: (i, 0, 0)),
            pl.BlockSpec((RB, K, C), lambda i: (i, 0, 0)),
            pl.BlockSpec((1, CS), lambda i: (0, 0)),
            pl.BlockSpec((1, CS), lambda i: (0, 0)),
            pl.BlockSpec((CS, CS), lambda i: (0, 0)),
            pl.BlockSpec((1, CS), lambda i: (0, 0)),
            pl.BlockSpec((3, 3), lambda i: (0, 0)),
            pl.BlockSpec((1, 3), lambda i: (0, 0)),
            pl.BlockSpec((1, 3), lambda i: (0, 0)),
            pl.BlockSpec((1, 3), lambda i: (0, 0)),
            pl.BlockSpec((3, C), lambda i: (0, 0)),
            pl.BlockSpec((1, C), lambda i: (0, 0)),
        ],
        out_specs=pl.BlockSpec((RB, C), lambda i: (i, 0)),
        out_shape=jax.ShapeDtypeStruct((N, C), jnp.float32),
    )(w2, gx, gv, s3, t3, Ww2, bw22, Wp1, bp12, s1, t1, Wp2, bp22)


def _bn_scale_shift(st, gamma, beta, width):
    cnt = jnp.float32(M)
    mean = st[0, 0:width] / cnt
    var = st[1, 0:width] / cnt - mean * mean
    s = gamma / jnp.sqrt(var + 1e-5)
    t = beta - mean * s
    return s.reshape(1, width), t.reshape(1, width)


def kernel(p, x, o, edges, boundary, Wq, bq, Wk, bk, Wv, bv, Wp1, bp1, g_p,
           b_p, Wp2, bp2, g_w1, b_w1, Ww1, bw1, g_w2, b_w2, Ww2, bw2):
    del o, edges, boundary
    pT = p.T
    idx, x_k, x_v = _knn(p, pT, x, Wk, bk.reshape(1, C), Wv, bv.reshape(1, C))

    p_pad = jnp.pad(p, ((0, 0), (0, C - 3)))
    gk, gv, gp = _sc_gather_call(x_k, x_v, p_pad, idx.reshape(M))
    gk = gk.reshape(N, K, C)
    gv = gv.reshape(N, K, C)
    gp = gp.reshape(N, K, C)

    bp12 = bp1.reshape(1, 3)
    bp22 = bp2.reshape(1, C)
    gx, st1 = _bn1_stats(gp, p, Wp1, bp12)
    s1, t1 = _bn_scale_shift(st1, g_p, b_p, 3)

    w1, st2 = _w1_pass(gx, gk, x, Wq, bq.reshape(1, C),
                       Wp1, bp12, s1, t1, Wp2, bp22)
    s2, t2 = _bn_scale_shift(st2, g_w1, b_w1, C)

    w2, st3 = _w2_pass(w1, s2, t2, Ww1, bw1.reshape(1, CS))
    s3, t3 = _bn_scale_shift(st3, g_w2, b_w2, CS)

    return _out_pass(w2, gx, gv, s3, t3, Ww2, bw2.reshape(1, CS),
                     Wp1, bp12, s1, t1, Wp2, bp22)


# speedup vs baseline: 5.1739x; 1.0480x over previous
import functools

import jax
import jax.numpy as jnp
from jax import lax
from jax.experimental import pallas as pl
from jax.experimental.pallas import tpu as pltpu
from jax.experimental.pallas import tpu_sc as plsc

N = 10000
K = 16
C = 128
CS = 16
SH = 8
RQ = 200
RB = 400
M = N * K

_PC = functools.partial(pl.pallas_call)


def _knn_body(pq_ref, pt_ref, x_ref, wk_ref, bk_ref, wv_ref, bv_ref,
              idx_ref, xk_ref, xv_ref, d_ref):
    pq = pq_ref[...]
    pt = pt_ref[...]
    sqq = jnp.sum(pq * pq, axis=1, keepdims=True)
    sqa = jnp.sum(pt * pt, axis=0, keepdims=True)
    cross = _dot_bf(pq, pt)
    d_ref[...] = sqq + sqa - 2.0 * cross

    lanes = lax.broadcasted_iota(jnp.int32, (RQ, N), 1)
    for t in range(K):
        d = d_ref[...]
        sel = jnp.argmin(d, axis=1).astype(jnp.int32)[:, None]
        idx_ref[:, t:t + 1] = sel
        d_ref[...] = jnp.where(lanes == sel, jnp.float32(jnp.inf), d)

    x = x_ref[...]
    xk_ref[...] = _dot_bf(x, wk_ref[...]) + bk_ref[...]
    xv_ref[...] = _dot_bf(x, wv_ref[...]) + bv_ref[...]


def _knn(p, pT, x, Wk, bk2, Wv, bv2):
    return _PC(
        _knn_body,
        grid=(N // RQ,),
        in_specs=[
            pl.BlockSpec((RQ, 3), lambda i: (i, 0)),
            pl.BlockSpec((3, N), lambda i: (0, 0)),
            pl.BlockSpec((RQ, C), lambda i: (i, 0)),
            pl.BlockSpec((C, C), lambda i: (0, 0)),
            pl.BlockSpec((1, C), lambda i: (0, 0)),
            pl.BlockSpec((C, C), lambda i: (0, 0)),
            pl.BlockSpec((1, C), lambda i: (0, 0)),
        ],
        out_specs=[
            pl.BlockSpec((RQ, K), lambda i: (i, 0)),
            pl.BlockSpec((RQ, C), lambda i: (i, 0)),
            pl.BlockSpec((RQ, C), lambda i: (i, 0)),
        ],
        out_shape=[
            jax.ShapeDtypeStruct((N, K), jnp.int32),
            jax.ShapeDtypeStruct((N, C), jnp.float32),
            jax.ShapeDtypeStruct((N, C), jnp.float32),
        ],
        scratch_shapes=[pltpu.VMEM((RQ, N), jnp.float32)],
    )(p, pT, x, Wk, bk2, Wv, bv2)


_NW = 32
_BPW = M // _NW
_CH = 200
_NCH = _BPW // _CH


def _sc_gather_body(xk_hbm, xv_hbm, pp_hbm, idx_hbm,
                    gk_hbm, gv_hbm, gp_hbm,
                    idx_v, rows_v, prow_v, sem):
    wid = lax.axis_index("s") * 2 + lax.axis_index("c")
    base = wid * _BPW
    for i in range(_NCH):
        off = base + i * _CH
        pltpu.sync_copy(idx_hbm.at[pl.ds(off, _CH)], idx_v)
        pltpu.async_copy(xk_hbm.at[idx_v], rows_v, sem).wait()
        pltpu.sync_copy(rows_v, gk_hbm.at[pl.ds(off, _CH)])
        pltpu.async_copy(xv_hbm.at[idx_v], rows_v, sem).wait()
        pltpu.sync_copy(rows_v, gv_hbm.at[pl.ds(off, _CH)])
        pltpu.async_copy(pp_hbm.at[idx_v], prow_v, sem).wait()
        pltpu.sync_copy(prow_v, gp_hbm.at[pl.ds(off, _CH)])


def _sc_gather_call(x_k, x_v, p_pad, idx_flat):
    mesh = plsc.VectorSubcoreMesh(core_axis_name="c", subcore_axis_name="s")
    fn = pl.kernel(
        _sc_gather_body,
        mesh=mesh,
        out_type=[
            jax.ShapeDtypeStruct((M, C), jnp.float32),
            jax.ShapeDtypeStruct((M, C), jnp.float32),
            jax.ShapeDtypeStruct((M, C), jnp.float32),
        ],
        scratch_types=[
            pltpu.VMEM((_CH,), jnp.int32),
            pltpu.VMEM((_CH, C), jnp.float32),
            pltpu.VMEM((_CH, C), jnp.float32),
            pltpu.SemaphoreType.DMA,
        ],
    )
    return fn(x_k, x_v, p_pad, idx_flat)


def _dot_bf(a, b):
    return jnp.dot(a.astype(jnp.bfloat16), b.astype(jnp.bfloat16),
                   preferred_element_type=jnp.float32)


def _mm3(a2, w_ref, b_ref):
    return _dot_bf(a2, w_ref[...]) + b_ref[...]


def _p_r(gx3, wp1_ref, bp1_ref, s1_ref, t1_ref, wp2_ref, bp2_ref):
    gxyz = gx3[:, :, 0:3].reshape(RB * K, 3)
    pr0 = _mm3(gxyz, wp1_ref, bp1_ref)
    prb = jnp.maximum(pr0 * s1_ref[...] + t1_ref[...], 0.0)
    return _mm3(prb, wp2_ref, bp2_ref)


def _bn1_body(gp_ref, p_ref, wp1_ref, bp1_ref, gx_ref, st_ref):
    gxyz3 = gp_ref[...][:, :, 0:3] - p_ref[...][:, None, :]
    gx_ref[...] = jnp.concatenate(
        [gxyz3, jnp.zeros((RB, K, 13), jnp.float32)], axis=2)
    pr0 = _mm3(gxyz3.reshape(RB * K, 3), wp1_ref, bp1_ref)
    s = jnp.sum(pr0, axis=0, keepdims=True)
    q = jnp.sum(pr0 * pr0, axis=0, keepdims=True)

    @pl.when(pl.program_id(0) == 0)
    def _():
        st_ref[...] = jnp.zeros_like(st_ref)

    st_ref[0:1, 0:3] += s
    st_ref[1:2, 0:3] += q


def _bn1_stats(gp, p, Wp1, bp12):
    return _PC(
        _bn1_body,
        grid=(N // RB,),
        in_specs=[
            pl.BlockSpec((RB, K, C), lambda i: (i, 0, 0)),
            pl.BlockSpec((RB, 3), lambda i: (i, 0)),
            pl.BlockSpec((3, 3), lambda i: (0, 0)),
            pl.BlockSpec((1, 3), lambda i: (0, 0)),
        ],
        out_specs=[
            pl.BlockSpec((RB, K, 16), lambda i: (i, 0, 0)),
            pl.BlockSpec((2, 128), lambda i: (0, 0)),
        ],
        out_shape=[
            jax.ShapeDtypeStruct((N, K, 16), jnp.float32),
            jax.ShapeDtypeStruct((2, 128), jnp.float32),
        ],
    )(gp, p, Wp1, bp12)


def _w1_body(gx_ref, gk_ref, x_ref, wq_ref, bq_ref,
             wp1_ref, bp1_ref, s1_ref, t1_ref, wp2_ref, bp2_ref,
             w1_ref, st_ref):
    xq = _dot_bf(x_ref[...], wq_ref[...]) + bq_ref[...]
    p_r = _p_r(gx_ref[...], wp1_ref, bp1_ref,
               s1_ref, t1_ref, wp2_ref, bp2_ref)
    gk2 = gk_ref[...].reshape(RB * K, C)
    xqr = jnp.broadcast_to(xq[:, None, :], (RB, K, C)).reshape(RB * K, C)
    w1 = gk2 - xqr + p_r
    w1_ref[...] = w1.reshape(RB, K, C)

    @pl.when(pl.program_id(0) == 0)
    def _():
        st_ref[...] = jnp.zeros_like(st_ref)

    st_ref[0:1, :] += jnp.sum(w1, axis=0, keepdims=True)
    st_ref[1:2, :] += jnp.sum(w1 * w1, axis=0, keepdims=True)


def _w1_pass(gx, gk, x, Wq, bq2, Wp1, bp12, s1, t1, Wp2, bp22):
    return _PC(
        _w1_body,
        grid=(N // RB,),
        in_specs=[
            pl.BlockSpec((RB, K, 16), lambda i: (i, 0, 0)),
            pl.BlockSpec((RB, K, C), lambda i: (i, 0, 0)),
            pl.BlockSpec((RB, C), lambda i: (i, 0)),
            pl.BlockSpec((C, C), lambda i: (0, 0)),
            pl.BlockSpec((1, C), lambda i: (0, 0)),
            pl.BlockSpec((3, 3), lambda i: (0, 0)),
            pl.BlockSpec((1, 3), lambda i: (0, 0)),
            pl.BlockSpec((1, 3), lambda i: (0, 0)),
            pl.BlockSpec((1, 3), lambda i: (0, 0)),
            pl.BlockSpec((3, C), lambda i: (0, 0)),
            pl.BlockSpec((1, C), lambda i: (0, 0)),
        ],
        out_specs=[
            pl.BlockSpec((RB, K, C), lambda i: (i, 0, 0)),
            pl.BlockSpec((2, C), lambda i: (0, 0)),
        ],
        out_shape=[
            jax.ShapeDtypeStruct((N, K, C), jnp.float32),
            jax.ShapeDtypeStruct((2, C), jnp.float32),
        ],
    )(gx, gk, x, Wq, bq2, Wp1, bp12, s1, t1, Wp2, bp22)


def _w2_body(w1_ref, s2_ref, t2_ref, ww1_ref, bw1_ref, w2_ref, st_ref):
    h = jnp.maximum(w1_ref[...].reshape(RB * K, C) * s2_ref[...]
                    + t2_ref[...], 0.0)
    w2 = _dot_bf(h, ww1_ref[...]) + bw1_ref[...]
    w2_ref[...] = w2.reshape(RB, K, CS)

    @pl.when(pl.program_id(0) == 0)
    def _():
        st_ref[...] = jnp.zeros_like(st_ref)

    st_ref[0:1, 0:CS] += jnp.sum(w2, axis=0, keepdims=True)
    st_ref[1:2, 0:CS] += jnp.sum(w2 * w2, axis=0, keepdims=True)


def _w2_pass(w1, s2, t2, Ww1, bw12):
    return _PC(
        _w2_body,
        grid=(N // RB,),
        in_specs=[
            pl.BlockSpec((RB, K, C), lambda i: (i, 0, 0)),
            pl.BlockSpec((1, C), lambda i: (0, 0)),
            pl.BlockSpec((1, C), lambda i: (0, 0)),
            pl.BlockSpec((C, CS), lambda i: (0, 0)),
            pl.BlockSpec((1, CS), lambda i: (0, 0)),
        ],
        out_specs=[
            pl.BlockSpec((RB, K, CS), lambda i: (i, 0, 0)),
            pl.BlockSpec((2, 128), lambda i: (0, 0)),
        ],
        out_shape=[
            jax.ShapeDtypeStruct((N, K, CS), jnp.float32),
            jax.ShapeDtypeStruct((2, 128), jnp.float32),
        ],
    )(w1, s2, t2, Ww1, bw12)


def _out_body(w2_ref, gx_ref, gv_ref, s3_ref, t3_ref, ww2_ref, bw2_ref,
              wp1_ref, bp1_ref, s1_ref, t1_ref, wp2_ref, bp2_ref, out_ref):
    h2 = jnp.maximum(w2_ref[...].reshape(RB * K, CS) * s3_ref[...]
                     + t3_ref[...], 0.0)
    a = (_dot_bf(h2, ww2_ref[...]) + bw2_ref[...]).reshape(RB, K, CS)
    mx = jnp.max(a, axis=1, keepdims=True)
    e = jnp.exp(a - mx)
    w = e / jnp.sum(e, axis=1, keepdims=True)
    wfull = jnp.concatenate([w] * SH, axis=2)
    p_r = _p_r(gx_ref[...], wp1_ref, bp1_ref,
               s1_ref, t1_ref, wp2_ref, bp2_ref).reshape(RB, K, C)
    out_ref[...] = jnp.sum((gv_ref[...] + p_r) * wfull, axis=1)


def _out_pass(w2, gx, gv, s3, t3, Ww2, bw22, Wp1, bp12, s1, t1, Wp2, bp22):
    return _PC(
        _out_body,
        grid=(N // RB,),
        in_specs=[
            pl.BlockSpec((RB, K, CS), lambda i: (i, 0, 0)),
            pl.BlockSpec((RB, K, 16), lambda i: (i, 0, 0)),
            pl.BlockSpec((RB, K, C), lambda i: (i, 0, 0)),
            pl.BlockSpec((1, CS), lambda i: (0, 0)),
            pl.BlockSpec((1, CS), lambda i: (0, 0)),
            pl.BlockSpec((CS, CS), lambda i: (0, 0)),
            pl.BlockSpec((1, CS), lambda i: (0, 0)),
            pl.BlockSpec((3, 3), lambda i: (0, 0)),
            pl.BlockSpec((1, 3), lambda i: (0, 0)),
            pl.BlockSpec((1, 3), lambda i: (0, 0)),
            pl.BlockSpec((1, 3), lambda i: (0, 0)),
            pl.BlockSpec((3, C), lambda i: (0, 0)),
            pl.BlockSpec((1, C), lambda i: (0, 0)),
        ],
        out_specs=pl.BlockSpec((RB, C), lambda i: (i, 0)),
        out_shape=jax.ShapeDtypeStruct((N, C), jnp.float32),
    )(w2, gx, gv, s3, t3, Ww2, bw22, Wp1, bp12, s1, t1, Wp2, bp22)


def _bn_scale_shift(st, gamma, beta, width):
    cnt = jnp.float32(M)
    mean = st[0, 0:width] / cnt
    var = st[1, 0:width] / cnt - mean * mean
    s = gamma / jnp.sqrt(var + 1e-5)
    t = beta - mean * s
    return s.reshape(1, width), t.reshape(1, width)


def kernel(p, x, o, edges, boundary, Wq, bq, Wk, bk, Wv, bv, Wp1, bp1, g_p,
           b_p, Wp2, bp2, g_w1, b_w1, Ww1, bw1, g_w2, b_w2, Ww2, bw2):
    del o, edges, boundary
    pT = p.T
    idx, x_k, x_v = _knn(p, pT, x, Wk, bk.reshape(1, C), Wv, bv.reshape(1, C))

    p_pad = jnp.pad(p, ((0, 0), (0, C - 3)))
    gk, gv, gp = _sc_gather_call(x_k, x_v, p_pad, idx.reshape(M))
    gk = gk.reshape(N, K, C)
    gv = gv.reshape(N, K, C)
    gp = gp.reshape(N, K, C)

    bp12 = bp1.reshape(1, 3)
    bp22 = bp2.reshape(1, C)
    gx, st1 = _bn1_stats(gp, p, Wp1, bp12)
    s1, t1 = _bn_scale_shift(st1, g_p, b_p, 3)

    w1, st2 = _w1_pass(gx, gk, x, Wq, bq.reshape(1, C),
                       Wp1, bp12, s1, t1, Wp2, bp22)
    s2, t2 = _bn_scale_shift(st2, g_w1, b_w1, C)

    w2, st3 = _w2_pass(w1, s2, t2, Ww1, bw1.reshape(1, CS))
    s3, t3 = _bn_scale_shift(st3, g_w2, b_w2, CS)

    return _out_pass(w2, gx, gv, s3, t3, Ww2, bw2.reshape(1, CS),
                     Wp1, bp12, s1, t1, Wp2, bp22)
